# baseline (speedup 1.0000x reference)
import functools

import jax
import jax.numpy as jnp
from jax import lax
from jax.experimental import pallas as pl
from jax.experimental.pallas import tpu as pltpu
from jax.experimental.pallas import tpu_sc as plsc

N_RNA = 1927
N = 10000
D = 128
E = 320000
EPS = 1e-5

NPAD = 10240
EPAD = 327680
EROWS = EPAD // 128
TROWS = EROWS // 32
ACC_ROWS = NPAD // 16

_MESH = plsc.VectorSubcoreMesh(
    core_axis_name="c", subcore_axis_name="s", num_cores=2, num_subcores=16)


@functools.partial(
    pl.kernel,
    out_type=jax.ShapeDtypeStruct((2, NPAD, 16), jnp.float32),
    mesh=_MESH,
    scratch_types=[
        pltpu.VMEM((TROWS, 128), jnp.int32),
        pltpu.VMEM((128, 16), jnp.float32),
        pltpu.VMEM_SHARED((NPAD, 16), jnp.float32),
    ],
)
def _deg_kernel(dst_hbm, out_hbm, idx_v, buf, acc):
    cid = lax.axis_index("c")
    sid = lax.axis_index("s")
    wid = sid * 2 + cid

    z16 = jnp.zeros((16,), jnp.float32)

    def zrow(i, _):
        buf[i, :] = z16
        return 0

    lax.fori_loop(0, 128, zrow, 0)
    for j in range(ACC_ROWS // 128):
        pltpu.sync_copy(buf, acc.at[pl.ds(sid * ACC_ROWS + j * 128, 128)])

    o16 = jnp.ones((16,), jnp.float32)

    def orow(i, _):
        buf[i, :] = o16
        return 0

    lax.fori_loop(0, 128, orow, 0)
    plsc.subcore_barrier()

    pltpu.sync_copy(dst_hbm.at[pl.ds(wid * TROWS, TROWS)], idx_v)

    def edge_chunk(i, _):
        pltpu.sync_copy(buf, acc.at[idx_v.at[i]], add=True)
        return 0

    lax.fori_loop(0, TROWS, edge_chunk, 0)
    plsc.subcore_barrier()

    pltpu.sync_copy(
        acc.at[pl.ds(sid * ACC_ROWS, ACC_ROWS)],
        out_hbm.at[cid, pl.ds(sid * ACC_ROWS, ACC_ROWS)])


@functools.partial(
    pl.kernel,
    out_type=jax.ShapeDtypeStruct((2, NPAD, D), jnp.float32),
    mesh=_MESH,
    scratch_types=[
        pltpu.VMEM((TROWS, 128), jnp.int32),
        pltpu.VMEM((TROWS, 128), jnp.int32),
        pltpu.VMEM((128, D), jnp.float32),
        pltpu.VMEM_SHARED((NPAD, D), jnp.float32),
        pltpu.SemaphoreType.DMA,
    ],
)
def _agg_kernel(g_hbm, src_hbm, dst_hbm, out_hbm, src_v, dst_v, buf0, acc,
                gs0):
    cid = lax.axis_index("c")
    sid = lax.axis_index("s")
    wid = sid * 2 + cid

    z16 = jnp.zeros((16,), jnp.float32)

    def zrow(i, _):
        def zcol(k, _):
            buf0[i, pl.ds(k * 16, 16)] = z16
            return 0
        lax.fori_loop(0, D // 16, zcol, 0)
        return 0

    lax.fori_loop(0, 128, zrow, 0)
    for j in range(ACC_ROWS // 128):
        pltpu.sync_copy(buf0, acc.at[pl.ds(sid * ACC_ROWS + j * 128, 128)])
    plsc.subcore_barrier()

    pltpu.sync_copy(src_hbm.at[pl.ds(wid * TROWS, TROWS)], src_v)
    pltpu.sync_copy(dst_hbm.at[pl.ds(wid * TROWS, TROWS)], dst_v)

    def edge_chunk(i, _):
        pltpu.async_copy(g_hbm.at[src_v.at[i]], buf0, gs0).wait()
        pltpu.sync_copy(buf0, acc.at[dst_v.at[i]], add=True)
        return 0

    lax.fori_loop(0, TROWS, edge_chunk, 0)
    plsc.subcore_barrier()

    pltpu.sync_copy(
        acc.at[pl.ds(sid * ACC_ROWS, ACC_ROWS)],
        out_hbm.at[cid, pl.ds(sid * ACC_ROWS, ACC_ROWS)])


_BM = 256
_GRID = NPAD // _BM


def _dis_from_deg(degT_blk):
    deg = jnp.sum(degT_blk, axis=1, keepdims=True) * (1.0 / 16.0) + 1.0
    return lax.rsqrt(deg)


def _mm_scale_body(x_ref, w_ref, degT_ref, o_ref):
    dis = _dis_from_deg(degT_ref[...])
    o_ref[...] = jnp.dot(
        x_ref[...], w_ref[...], preferred_element_type=jnp.float32) * dis


def _mm_scale(x, w, degT):
    return pl.pallas_call(
        _mm_scale_body,
        grid=(_GRID,),
        in_specs=[
            pl.BlockSpec((_BM, D), lambda i: (i, 0)),
            pl.BlockSpec((D, D), lambda i: (0, 0)),
            pl.BlockSpec((_BM, 32), lambda i: (i, 0)),
        ],
        out_specs=pl.BlockSpec((_BM, D), lambda i: (i, 0)),
        out_shape=jax.ShapeDtypeStruct((NPAD, D), jnp.float32),
    )(x, w, degT)


def _combine_stats_body(p0_ref, p1_ref, g_ref, degT_ref, b_ref,
                        t_ref, sum_ref, sq_ref):
    i = pl.program_id(0)
    dis = _dis_from_deg(degT_ref[...])
    t = (p0_ref[0] + p1_ref[0] + g_ref[...]) * dis + b_ref[...]
    t_ref[...] = t
    row = i * _BM + lax.broadcasted_iota(jnp.int32, (_BM, 1), 0)
    tm = jnp.where(row < N, t, 0.0)

    @pl.when(i == 0)
    def _():
        sum_ref[...] = jnp.zeros_like(sum_ref)
        sq_ref[...] = jnp.zeros_like(sq_ref)

    sum_ref[...] += jnp.sum(tm, axis=0, keepdims=True)
    sq_ref[...] += jnp.sum(tm * tm, axis=0, keepdims=True)


def _combine_stats(p, g, degT, b):
    return pl.pallas_call(
        _combine_stats_body,
        grid=(_GRID,),
        in_specs=[
            pl.BlockSpec((1, _BM, D), lambda i: (0, i, 0)),
            pl.BlockSpec((1, _BM, D), lambda i: (1, i, 0)),
            pl.BlockSpec((_BM, D), lambda i: (i, 0)),
            pl.BlockSpec((_BM, 32), lambda i: (i, 0)),
            pl.BlockSpec((1, D), lambda i: (0, 0)),
        ],
        out_specs=[
            pl.BlockSpec((_BM, D), lambda i: (i, 0)),
            pl.BlockSpec((1, D), lambda i: (0, 0)),
            pl.BlockSpec((1, D), lambda i: (0, 0)),
        ],
        out_shape=[
            jax.ShapeDtypeStruct((NPAD, D), jnp.float32),
            jax.ShapeDtypeStruct((1, D), jnp.float32),
            jax.ShapeDtypeStruct((1, D), jnp.float32),
        ],
    )(p, p, g, degT, b)


def _bn_relu_mm_body(t_ref, s_ref, q_ref, gam_ref, bet_ref, w_ref, degT_ref,
                     o_ref):
    i = pl.program_id(0)
    mean = s_ref[...] / N
    var = q_ref[...] / N - mean * mean
    inv = lax.rsqrt(var + EPS)
    y = (t_ref[...] - mean) * inv * gam_ref[...] + bet_ref[...]
    y = jnp.maximum(y, 0.0)
    row = i * _BM + lax.broadcasted_iota(jnp.int32, (_BM, 1), 0)
    dis = jnp.where(row < N, _dis_from_deg(degT_ref[...]), 0.0)
    o_ref[...] = jnp.dot(
        y * dis, w_ref[...], preferred_element_type=jnp.float32)


def _bn_relu_mm(t, s, q, gamma, beta, w, degT):
    return pl.pallas_call(
        _bn_relu_mm_body,
        grid=(_GRID,),
        in_specs=[
            pl.BlockSpec((_BM, D), lambda i: (i, 0)),
            pl.BlockSpec((1, D), lambda i: (0, 0)),
            pl.BlockSpec((1, D), lambda i: (0, 0)),
            pl.BlockSpec((1, D), lambda i: (0, 0)),
            pl.BlockSpec((1, D), lambda i: (0, 0)),
            pl.BlockSpec((D, D), lambda i: (0, 0)),
            pl.BlockSpec((_BM, 32), lambda i: (i, 0)),
        ],
        out_specs=pl.BlockSpec((_BM, D), lambda i: (i, 0)),
        out_shape=jax.ShapeDtypeStruct((NPAD, D), jnp.float32),
    )(t, s, q, gamma, beta, w, degT)


def _bn_final_body(t_ref, s_ref, q_ref, gam_ref, bet_ref, o_ref):
    mean = s_ref[...] / N
    var = q_ref[...] / N - mean * mean
    inv = lax.rsqrt(var + EPS)
    o_ref[...] = (t_ref[...] - mean) * inv * gam_ref[...] + bet_ref[...]


def _bn_final(t, s, q, gamma, beta):
    return pl.pallas_call(
        _bn_final_body,
        grid=(_GRID,),
        in_specs=[
            pl.BlockSpec((_BM, D), lambda i: (i, 0)),
            pl.BlockSpec((1, D), lambda i: (0, 0)),
            pl.BlockSpec((1, D), lambda i: (0, 0)),
            pl.BlockSpec((1, D), lambda i: (0, 0)),
            pl.BlockSpec((1, D), lambda i: (0, 0)),
        ],
        out_specs=pl.BlockSpec((_BM, D), lambda i: (i, 0)),
        out_shape=jax.ShapeDtypeStruct((NPAD, D), jnp.float32),
    )(t, s, q, gamma, beta)


def kernel(x, W1, b1, gamma1, beta1, W2, b2, gamma2, beta2, edge_index):
    pad_e = EPAD - E
    src = jnp.concatenate(
        [edge_index[0], jnp.full((pad_e,), N, jnp.int32)])
    dst = jnp.concatenate(
        [edge_index[1], jnp.full((pad_e,), N, jnp.int32)])
    src2d = src.reshape(EROWS, 128)
    dst2d = dst.reshape(EROWS, 128)

    x_p = jnp.pad(x, ((0, NPAD - N), (0, 0)))
    b1r = b1.reshape(1, D)
    b2r = b2.reshape(1, D)
    g1r = gamma1.reshape(1, D)
    g2r = gamma2.reshape(1, D)
    be1 = beta1.reshape(1, D)
    be2 = beta2.reshape(1, D)

    deg_parts = _deg_kernel(dst2d)
    degT = deg_parts.transpose(1, 0, 2).reshape(NPAD, 32)

    g1 = _mm_scale(x_p, W1, degT)
    p1 = _agg_kernel(g1, src2d, dst2d)
    t1, s1, q1 = _combine_stats(p1, g1, degT, b1r)
    g2 = _bn_relu_mm(t1, s1, q1, g1r, be1, W2, degT)
    p2 = _agg_kernel(g2, src2d, dst2d)
    t2, s2, q2 = _combine_stats(p2, g2, degT, b2r)
    out = _bn_final(t2, s2, q2, g2r, be2)

    h = out[:N]
    return (h[:N_RNA], h[N_RNA:])


# speedup vs baseline: 1.1536x; 1.1536x over previous
import functools

import jax
import jax.numpy as jnp
from jax import lax
from jax.experimental import pallas as pl
from jax.experimental.pallas import tpu as pltpu
from jax.experimental.pallas import tpu_sc as plsc

N_RNA = 1927
N = 10000
D = 128
E = 320000
EPS = 1e-5

NPAD = 10240
EPAD = 327680
EROWS = EPAD // 128
TROWS = EROWS // 32
ACC_ROWS = NPAD // 16

_MESH = plsc.VectorSubcoreMesh(
    core_axis_name="c", subcore_axis_name="s", num_cores=2, num_subcores=16)


@functools.partial(
    pl.kernel,
    out_type=jax.ShapeDtypeStruct((2, NPAD, D), jnp.float32),
    mesh=_MESH,
    scratch_types=[
        pltpu.VMEM((TROWS, 128), jnp.int32),
        pltpu.VMEM((128, D), jnp.float32),
        pltpu.VMEM_SHARED((NPAD, D), jnp.float32),
    ],
)
def _deg_kernel(dst_hbm, out_hbm, idx_v, buf, acc):
    cid = lax.axis_index("c")
    sid = lax.axis_index("s")
    wid = sid * 2 + cid

    z16 = jnp.zeros((16,), jnp.float32)

    def zrow(i, _):
        def zcol(k, _):
            buf[i, pl.ds(k * 16, 16)] = z16
            return 0
        lax.fori_loop(0, D // 16, zcol, 0)
        return 0

    lax.fori_loop(0, 128, zrow, 0)
    for j in range(ACC_ROWS // 128):
        pltpu.sync_copy(buf, acc.at[pl.ds(sid * ACC_ROWS + j * 128, 128)])

    o16 = jnp.ones((16,), jnp.float32)

    def orow(i, _):
        def ocol(k, _):
            buf[i, pl.ds(k * 16, 16)] = o16
            return 0
        lax.fori_loop(0, D // 16, ocol, 0)
        return 0

    lax.fori_loop(0, 128, orow, 0)
    plsc.subcore_barrier()

    pltpu.sync_copy(dst_hbm.at[pl.ds(wid * TROWS, TROWS)], idx_v)

    def edge_chunk(i, _):
        pltpu.sync_copy(buf, acc.at[idx_v.at[i]], add=True)
        return 0

    lax.fori_loop(0, TROWS, edge_chunk, 0)
    plsc.subcore_barrier()

    pltpu.sync_copy(
        acc.at[pl.ds(sid * ACC_ROWS, ACC_ROWS)],
        out_hbm.at[cid, pl.ds(sid * ACC_ROWS, ACC_ROWS)])


@functools.partial(
    pl.kernel,
    out_type=jax.ShapeDtypeStruct((2, NPAD, D), jnp.float32),
    mesh=_MESH,
    scratch_types=[
        pltpu.VMEM((TROWS, 128), jnp.int32),
        pltpu.VMEM((TROWS, 128), jnp.int32),
        pltpu.VMEM((128, D), jnp.float32),
        pltpu.VMEM_SHARED((NPAD, D), jnp.float32),
        pltpu.SemaphoreType.DMA,
    ],
)
def _agg_kernel(g_hbm, src_hbm, dst_hbm, out_hbm, src_v, dst_v, buf0, acc,
                gs0):
    cid = lax.axis_index("c")
    sid = lax.axis_index("s")
    wid = sid * 2 + cid

    z16 = jnp.zeros((16,), jnp.float32)

    def zrow(i, _):
        def zcol(k, _):
            buf0[i, pl.ds(k * 16, 16)] = z16
            return 0
        lax.fori_loop(0, D // 16, zcol, 0)
        return 0

    lax.fori_loop(0, 128, zrow, 0)
    for j in range(ACC_ROWS // 128):
        pltpu.sync_copy(buf0, acc.at[pl.ds(sid * ACC_ROWS + j * 128, 128)])
    plsc.subcore_barrier()

    pltpu.sync_copy(src_hbm.at[pl.ds(wid * TROWS, TROWS)], src_v)
    pltpu.sync_copy(dst_hbm.at[pl.ds(wid * TROWS, TROWS)], dst_v)

    def edge_chunk(i, _):
        pltpu.async_copy(g_hbm.at[src_v.at[i]], buf0, gs0).wait()
        pltpu.sync_copy(buf0, acc.at[dst_v.at[i]], add=True)
        return 0

    lax.fori_loop(0, TROWS, edge_chunk, 0)
    plsc.subcore_barrier()

    pltpu.sync_copy(
        acc.at[pl.ds(sid * ACC_ROWS, ACC_ROWS)],
        out_hbm.at[cid, pl.ds(sid * ACC_ROWS, ACC_ROWS)])


_BM = 256
_GRID = NPAD // _BM


def _dis_from_deg(d0, d1):
    deg = (d0 + d1)[:, 0:1] + 1.0
    return lax.rsqrt(deg)


def _mm_scale_body(x_ref, w_ref, d0_ref, d1_ref, o_ref):
    dis = _dis_from_deg(d0_ref[0], d1_ref[0])
    o_ref[...] = jnp.dot(
        x_ref[...], w_ref[...], preferred_element_type=jnp.float32) * dis


def _mm_scale(x, w, dp):
    return pl.pallas_call(
        _mm_scale_body,
        grid=(_GRID,),
        in_specs=[
            pl.BlockSpec((_BM, D), lambda i: (i, 0)),
            pl.BlockSpec((D, D), lambda i: (0, 0)),
            pl.BlockSpec((1, _BM, D), lambda i: (0, i, 0)),
            pl.BlockSpec((1, _BM, D), lambda i: (1, i, 0)),
        ],
        out_specs=pl.BlockSpec((_BM, D), lambda i: (i, 0)),
        out_shape=jax.ShapeDtypeStruct((NPAD, D), jnp.float32),
    )(x, w, dp, dp)


def _combine_stats_body(p0_ref, p1_ref, g_ref, d0_ref, d1_ref, b_ref,
                        t_ref, sum_ref, sq_ref):
    i = pl.program_id(0)
    dis = _dis_from_deg(d0_ref[0], d1_ref[0])
    t = (p0_ref[0] + p1_ref[0] + g_ref[...]) * dis + b_ref[...]
    t_ref[...] = t
    row = i * _BM + lax.broadcasted_iota(jnp.int32, (_BM, 1), 0)
    tm = jnp.where(row < N, t, 0.0)

    @pl.when(i == 0)
    def _():
        sum_ref[...] = jnp.zeros_like(sum_ref)
        sq_ref[...] = jnp.zeros_like(sq_ref)

    sum_ref[...] += jnp.sum(tm, axis=0, keepdims=True)
    sq_ref[...] += jnp.sum(tm * tm, axis=0, keepdims=True)


def _combine_stats(p, g, dp, b):
    return pl.pallas_call(
        _combine_stats_body,
        grid=(_GRID,),
        in_specs=[
            pl.BlockSpec((1, _BM, D), lambda i: (0, i, 0)),
            pl.BlockSpec((1, _BM, D), lambda i: (1, i, 0)),
            pl.BlockSpec((_BM, D), lambda i: (i, 0)),
            pl.BlockSpec((1, _BM, D), lambda i: (0, i, 0)),
            pl.BlockSpec((1, _BM, D), lambda i: (1, i, 0)),
            pl.BlockSpec((1, D), lambda i: (0, 0)),
        ],
        out_specs=[
            pl.BlockSpec((_BM, D), lambda i: (i, 0)),
            pl.BlockSpec((1, D), lambda i: (0, 0)),
            pl.BlockSpec((1, D), lambda i: (0, 0)),
        ],
        out_shape=[
            jax.ShapeDtypeStruct((NPAD, D), jnp.float32),
            jax.ShapeDtypeStruct((1, D), jnp.float32),
            jax.ShapeDtypeStruct((1, D), jnp.float32),
        ],
    )(p, p, g, dp, dp, b)


def _bn_relu_mm_body(t_ref, s_ref, q_ref, gam_ref, bet_ref, w_ref, d0_ref,
                     d1_ref, o_ref):
    i = pl.program_id(0)
    mean = s_ref[...] / N
    var = q_ref[...] / N - mean * mean
    inv = lax.rsqrt(var + EPS)
    y = (t_ref[...] - mean) * inv * gam_ref[...] + bet_ref[...]
    y = jnp.maximum(y, 0.0)
    row = i * _BM + lax.broadcasted_iota(jnp.int32, (_BM, 1), 0)
    dis = jnp.where(row < N, _dis_from_deg(d0_ref[0], d1_ref[0]), 0.0)
    o_ref[...] = jnp.dot(
        y * dis, w_ref[...], preferred_element_type=jnp.float32)


def _bn_relu_mm(t, s, q, gamma, beta, w, dp):
    return pl.pallas_call(
        _bn_relu_mm_body,
        grid=(_GRID,),
        in_specs=[
            pl.BlockSpec((_BM, D), lambda i: (i, 0)),
            pl.BlockSpec((1, D), lambda i: (0, 0)),
            pl.BlockSpec((1, D), lambda i: (0, 0)),
            pl.BlockSpec((1, D), lambda i: (0, 0)),
            pl.BlockSpec((1, D), lambda i: (0, 0)),
            pl.BlockSpec((D, D), lambda i: (0, 0)),
            pl.BlockSpec((1, _BM, D), lambda i: (0, i, 0)),
            pl.BlockSpec((1, _BM, D), lambda i: (1, i, 0)),
        ],
        out_specs=pl.BlockSpec((_BM, D), lambda i: (i, 0)),
        out_shape=jax.ShapeDtypeStruct((NPAD, D), jnp.float32),
    )(t, s, q, gamma, beta, w, dp, dp)


def _bn_final_body(t_ref, s_ref, q_ref, gam_ref, bet_ref, o_ref):
    mean = s_ref[...] / N
    var = q_ref[...] / N - mean * mean
    inv = lax.rsqrt(var + EPS)
    o_ref[...] = (t_ref[...] - mean) * inv * gam_ref[...] + bet_ref[...]


def _bn_final(t, s, q, gamma, beta):
    return pl.pallas_call(
        _bn_final_body,
        grid=(_GRID,),
        in_specs=[
            pl.BlockSpec((_BM, D), lambda i: (i, 0)),
            pl.BlockSpec((1, D), lambda i: (0, 0)),
            pl.BlockSpec((1, D), lambda i: (0, 0)),
            pl.BlockSpec((1, D), lambda i: (0, 0)),
            pl.BlockSpec((1, D), lambda i: (0, 0)),
        ],
        out_specs=pl.BlockSpec((_BM, D), lambda i: (i, 0)),
        out_shape=jax.ShapeDtypeStruct((NPAD, D), jnp.float32),
    )(t, s, q, gamma, beta)


def kernel(x, W1, b1, gamma1, beta1, W2, b2, gamma2, beta2, edge_index):
    pad_e = EPAD - E
    src = jnp.concatenate(
        [edge_index[0], jnp.full((pad_e,), N, jnp.int32)])
    dst = jnp.concatenate(
        [edge_index[1], jnp.full((pad_e,), N, jnp.int32)])
    src2d = src.reshape(EROWS, 128)
    dst2d = dst.reshape(EROWS, 128)

    x_p = jnp.pad(x, ((0, NPAD - N), (0, 0)))
    b1r = b1.reshape(1, D)
    b2r = b2.reshape(1, D)
    g1r = gamma1.reshape(1, D)
    g2r = gamma2.reshape(1, D)
    be1 = beta1.reshape(1, D)
    be2 = beta2.reshape(1, D)

    dp = _deg_kernel(dst2d)

    g1 = _mm_scale(x_p, W1, dp)
    p1 = _agg_kernel(g1, src2d, dst2d)
    t1, s1, q1 = _combine_stats(p1, g1, dp, b1r)
    g2 = _bn_relu_mm(t1, s1, q1, g1r, be1, W2, dp)
    p2 = _agg_kernel(g2, src2d, dst2d)
    t2, s2, q2 = _combine_stats(p2, g2, dp, b2r)
    out = _bn_final(t2, s2, q2, g2r, be2)

    h = out[:N]
    return (h[:N_RNA], h[N_RNA:])


# speedup vs baseline: 1.1686x; 1.0129x over previous
import functools

import jax
import jax.numpy as jnp
from jax import lax
from jax.experimental import pallas as pl
from jax.experimental.pallas import tpu as pltpu
from jax.experimental.pallas import tpu_sc as plsc

N_RNA = 1927
N = 10000
D = 128
E = 320000
EPS = 1e-5

NPAD = 10240
EPAD = 327680
EROWS = EPAD // 128
TROWS = EROWS // 32
ACC_ROWS = NPAD // 16

_MESH = plsc.VectorSubcoreMesh(
    core_axis_name="c", subcore_axis_name="s", num_cores=2, num_subcores=16)


@functools.partial(
    pl.kernel,
    out_type=jax.ShapeDtypeStruct((2, NPAD, D), jnp.float32),
    mesh=_MESH,
    scratch_types=[
        pltpu.VMEM((TROWS, 128), jnp.int32),
        pltpu.VMEM((128, D), jnp.float32),
        pltpu.VMEM_SHARED((NPAD, D), jnp.float32),
    ],
)
def _deg_kernel(dst_hbm, out_hbm, idx_v, buf, acc):
    cid = lax.axis_index("c")
    sid = lax.axis_index("s")
    wid = sid * 2 + cid

    z16 = jnp.zeros((16,), jnp.float32)

    def zrow(i, _):
        def zcol(k, _):
            buf[i, pl.ds(k * 16, 16)] = z16
            return 0
        lax.fori_loop(0, D // 16, zcol, 0)
        return 0

    lax.fori_loop(0, 128, zrow, 0)
    for j in range(ACC_ROWS // 128):
        pltpu.sync_copy(buf, acc.at[pl.ds(sid * ACC_ROWS + j * 128, 128)])

    o16 = jnp.ones((16,), jnp.float32)

    def orow(i, _):
        def ocol(k, _):
            buf[i, pl.ds(k * 16, 16)] = o16
            return 0
        lax.fori_loop(0, D // 16, ocol, 0)
        return 0

    lax.fori_loop(0, 128, orow, 0)
    plsc.subcore_barrier()

    pltpu.sync_copy(dst_hbm.at[pl.ds(wid * TROWS, TROWS)], idx_v)

    def edge_chunk(i, _):
        pltpu.sync_copy(buf, acc.at[idx_v.at[i]], add=True)
        return 0

    lax.fori_loop(0, TROWS, edge_chunk, 0)
    plsc.subcore_barrier()

    pltpu.sync_copy(
        acc.at[pl.ds(sid * ACC_ROWS, ACC_ROWS)],
        out_hbm.at[cid, pl.ds(sid * ACC_ROWS, ACC_ROWS)])


@functools.partial(
    pl.kernel,
    out_type=jax.ShapeDtypeStruct((2, NPAD, D), jnp.float32),
    mesh=_MESH,
    scratch_types=[
        pltpu.VMEM((40, 128), jnp.int32),
        pltpu.VMEM((40, 128), jnp.int32),
        pltpu.VMEM((128, D), jnp.float32),
        pltpu.VMEM((128, D), jnp.float32),
        pltpu.VMEM_SHARED((NPAD, D), jnp.float32),
        pltpu.SemaphoreType.DMA,
        pltpu.SemaphoreType.DMA,
    ],
)
def _agg_kernel(g_hbm, src_hbm, dst_hbm, out_hbm, src_v, dst_v, buf0, buf1,
                acc, gs0, gs1):
    cid = lax.axis_index("c")
    sid = lax.axis_index("s")
    wid = sid * 2 + cid

    z16 = jnp.zeros((16,), jnp.float32)

    def zrow(i, _):
        def zcol(k, _):
            buf0[i, pl.ds(k * 16, 16)] = z16
            return 0
        lax.fori_loop(0, D // 16, zcol, 0)
        return 0

    lax.fori_loop(0, 128, zrow, 0)
    for j in range(ACC_ROWS // 128):
        pltpu.sync_copy(buf0, acc.at[pl.ds(sid * ACC_ROWS + j * 128, 128)])
    plsc.subcore_barrier()

    def half_body(h, _):
        pltpu.sync_copy(
            src_hbm.at[pl.ds(wid * TROWS + h * 40, 40)], src_v)
        pltpu.sync_copy(
            dst_hbm.at[pl.ds(wid * TROWS + h * 40, 40)], dst_v)

        def pair(p, _):
            r = p * 2
            c0 = pltpu.async_copy(g_hbm.at[src_v.at[r]], buf0, gs0)
            c1 = pltpu.async_copy(g_hbm.at[src_v.at[r + 1]], buf1, gs1)
            c0.wait()
            pltpu.sync_copy(buf0, acc.at[dst_v.at[r]], add=True)
            c1.wait()
            pltpu.sync_copy(buf1, acc.at[dst_v.at[r + 1]], add=True)
            return 0

        lax.fori_loop(0, 20, pair, 0)
        return 0

    lax.fori_loop(0, 2, half_body, 0)
    plsc.subcore_barrier()

    pltpu.sync_copy(
        acc.at[pl.ds(sid * ACC_ROWS, ACC_ROWS)],
        out_hbm.at[cid, pl.ds(sid * ACC_ROWS, ACC_ROWS)])


_BM = 256
_GRID = NPAD // _BM


def _dis_from_deg(d0, d1):
    deg = (d0 + d1)[:, 0:1] + 1.0
    return lax.rsqrt(deg)


def _mm_scale_body(x_ref, w_ref, d0_ref, d1_ref, o_ref):
    dis = _dis_from_deg(d0_ref[0], d1_ref[0])
    o_ref[...] = jnp.dot(
        x_ref[...], w_ref[...], preferred_element_type=jnp.float32) * dis


def _mm_scale(x, w, dp):
    return pl.pallas_call(
        _mm_scale_body,
        grid=(_GRID,),
        in_specs=[
            pl.BlockSpec((_BM, D), lambda i: (i, 0)),
            pl.BlockSpec((D, D), lambda i: (0, 0)),
            pl.BlockSpec((1, _BM, D), lambda i: (0, i, 0)),
            pl.BlockSpec((1, _BM, D), lambda i: (1, i, 0)),
        ],
        out_specs=pl.BlockSpec((_BM, D), lambda i: (i, 0)),
        out_shape=jax.ShapeDtypeStruct((NPAD, D), jnp.float32),
    )(x, w, dp, dp)


def _combine_stats_body(p0_ref, p1_ref, g_ref, d0_ref, d1_ref, b_ref,
                        t_ref, sum_ref, sq_ref):
    i = pl.program_id(0)
    dis = _dis_from_deg(d0_ref[0], d1_ref[0])
    t = (p0_ref[0] + p1_ref[0] + g_ref[...]) * dis + b_ref[...]
    t_ref[...] = t
    row = i * _BM + lax.broadcasted_iota(jnp.int32, (_BM, 1), 0)
    tm = jnp.where(row < N, t, 0.0)

    @pl.when(i == 0)
    def _():
        sum_ref[...] = jnp.zeros_like(sum_ref)
        sq_ref[...] = jnp.zeros_like(sq_ref)

    sum_ref[...] += jnp.sum(tm, axis=0, keepdims=True)
    sq_ref[...] += jnp.sum(tm * tm, axis=0, keepdims=True)


def _combine_stats(p, g, dp, b):
    return pl.pallas_call(
        _combine_stats_body,
        grid=(_GRID,),
        in_specs=[
            pl.BlockSpec((1, _BM, D), lambda i: (0, i, 0)),
            pl.BlockSpec((1, _BM, D), lambda i: (1, i, 0)),
            pl.BlockSpec((_BM, D), lambda i: (i, 0)),
            pl.BlockSpec((1, _BM, D), lambda i: (0, i, 0)),
            pl.BlockSpec((1, _BM, D), lambda i: (1, i, 0)),
            pl.BlockSpec((1, D), lambda i: (0, 0)),
        ],
        out_specs=[
            pl.BlockSpec((_BM, D), lambda i: (i, 0)),
            pl.BlockSpec((1, D), lambda i: (0, 0)),
            pl.BlockSpec((1, D), lambda i: (0, 0)),
        ],
        out_shape=[
            jax.ShapeDtypeStruct((NPAD, D), jnp.float32),
            jax.ShapeDtypeStruct((1, D), jnp.float32),
            jax.ShapeDtypeStruct((1, D), jnp.float32),
        ],
    )(p, p, g, dp, dp, b)


def _bn_relu_mm_body(t_ref, s_ref, q_ref, gam_ref, bet_ref, w_ref, d0_ref,
                     d1_ref, o_ref):
    i = pl.program_id(0)
    mean = s_ref[...] / N
    var = q_ref[...] / N - mean * mean
    inv = lax.rsqrt(var + EPS)
    y = (t_ref[...] - mean) * inv * gam_ref[...] + bet_ref[...]
    y = jnp.maximum(y, 0.0)
    row = i * _BM + lax.broadcasted_iota(jnp.int32, (_BM, 1), 0)
    dis = jnp.where(row < N, _dis_from_deg(d0_ref[0], d1_ref[0]), 0.0)
    o_ref[...] = jnp.dot(
        y * dis, w_ref[...], preferred_element_type=jnp.float32)


def _bn_relu_mm(t, s, q, gamma, beta, w, dp):
    return pl.pallas_call(
        _bn_relu_mm_body,
        grid=(_GRID,),
        in_specs=[
            pl.BlockSpec((_BM, D), lambda i: (i, 0)),
            pl.BlockSpec((1, D), lambda i: (0, 0)),
            pl.BlockSpec((1, D), lambda i: (0, 0)),
            pl.BlockSpec((1, D), lambda i: (0, 0)),
            pl.BlockSpec((1, D), lambda i: (0, 0)),
            pl.BlockSpec((D, D), lambda i: (0, 0)),
            pl.BlockSpec((1, _BM, D), lambda i: (0, i, 0)),
            pl.BlockSpec((1, _BM, D), lambda i: (1, i, 0)),
        ],
        out_specs=pl.BlockSpec((_BM, D), lambda i: (i, 0)),
        out_shape=jax.ShapeDtypeStruct((NPAD, D), jnp.float32),
    )(t, s, q, gamma, beta, w, dp, dp)


def _bn_final_body(t_ref, s_ref, q_ref, gam_ref, bet_ref, o_ref):
    mean = s_ref[...] / N
    var = q_ref[...] / N - mean * mean
    inv = lax.rsqrt(var + EPS)
    o_ref[...] = (t_ref[...] - mean) * inv * gam_ref[...] + bet_ref[...]


def _bn_final(t, s, q, gamma, beta):
    return pl.pallas_call(
        _bn_final_body,
        grid=(_GRID,),
        in_specs=[
            pl.BlockSpec((_BM, D), lambda i: (i, 0)),
            pl.BlockSpec((1, D), lambda i: (0, 0)),
            pl.BlockSpec((1, D), lambda i: (0, 0)),
            pl.BlockSpec((1, D), lambda i: (0, 0)),
            pl.BlockSpec((1, D), lambda i: (0, 0)),
        ],
        out_specs=pl.BlockSpec((_BM, D), lambda i: (i, 0)),
        out_shape=jax.ShapeDtypeStruct((NPAD, D), jnp.float32),
    )(t, s, q, gamma, beta)


def kernel(x, W1, b1, gamma1, beta1, W2, b2, gamma2, beta2, edge_index):
    pad_e = EPAD - E
    src = jnp.concatenate(
        [edge_index[0], jnp.full((pad_e,), N, jnp.int32)])
    dst = jnp.concatenate(
        [edge_index[1], jnp.full((pad_e,), N, jnp.int32)])
    src2d = src.reshape(EROWS, 128)
    dst2d = dst.reshape(EROWS, 128)

    x_p = jnp.pad(x, ((0, NPAD - N), (0, 0)))
    b1r = b1.reshape(1, D)
    b2r = b2.reshape(1, D)
    g1r = gamma1.reshape(1, D)
    g2r = gamma2.reshape(1, D)
    be1 = beta1.reshape(1, D)
    be2 = beta2.reshape(1, D)

    dp = _deg_kernel(dst2d)

    g1 = _mm_scale(x_p, W1, dp)
    p1 = _agg_kernel(g1, src2d, dst2d)
    t1, s1, q1 = _combine_stats(p1, g1, dp, b1r)
    g2 = _bn_relu_mm(t1, s1, q1, g1r, be1, W2, dp)
    p2 = _agg_kernel(g2, src2d, dst2d)
    t2, s2, q2 = _combine_stats(p2, g2, dp, b2r)
    out = _bn_final(t2, s2, q2, g2r, be2)

    h = out[:N]
    return (h[:N_RNA], h[N_RNA:])
